# Initial kernel scaffold; baseline (speedup 1.0000x reference)
import functools

import jax
import jax.numpy as jnp
from jax import lax
from jax.experimental import pallas as pl
from jax.experimental.pallas import tpu as pltpu
from jax.experimental.pallas import tpu_sc as plsc

N = 50000
E = 800000
IN_CH = 128
HID_CH = 64
OUT_CH = 32
NUM_GRAPHS = 256

NC, NS, LANES = 2, 16, 16
NPAD = 50176
ROWS_PT = NPAD // NS
GP = 128
GPB = 10
BLK = GP * GPB
NBLK = E // BLK
ZR = 448
RB = 256
NRB = NPAD // RB


def _sc_degree_body(dst_hbm, ones_hbm, zeros_hbm, out_hbm,
                    didx, ones_v, zbuf, hist, sem):
    c = lax.axis_index("c")
    s = lax.axis_index("s")
    wid = c * NS + s

    pltpu.sync_copy(zeros_hbm, zbuf)
    base = s * ROWS_PT
    for r in range(ROWS_PT // ZR):
        pltpu.sync_copy(zbuf, hist.at[pl.ds(base + r * ZR, ZR)])
    pltpu.sync_copy(ones_hbm, ones_v)
    plsc.subcore_barrier()

    nblk = (NBLK - wid + (NC * NS - 1)) // (NC * NS)

    def blk(k, _):
        e0 = (wid + k * NC * NS) * BLK
        pltpu.sync_copy(dst_hbm.at[pl.ds(e0, BLK)], didx)
        descs = [
            pltpu.async_copy(ones_v, hist.at[didx.at[j]], sem, add=True)
            for j in range(GPB)
        ]
        for d in descs:
            d.wait()
        return 0

    lax.fori_loop(0, nblk, blk, 0)
    plsc.subcore_barrier()
    pltpu.sync_copy(hist.at[pl.ds(base, ROWS_PT)],
                    out_hbm.at[c].at[pl.ds(base, ROWS_PT)])


def _sc_degree(dst, ones_v, zeros_v):
    mesh = plsc.VectorSubcoreMesh(core_axis_name="c", subcore_axis_name="s",
                                  num_cores=NC, num_subcores=NS)
    k = pl.kernel(
        _sc_degree_body,
        out_type=jax.ShapeDtypeStruct((NC, NPAD), jnp.float32),
        mesh=mesh,
        scratch_types=[
            pltpu.VMEM((GPB, GP), jnp.int32),
            pltpu.VMEM((GP,), jnp.float32),
            pltpu.VMEM((ZR,), jnp.float32),
            pltpu.VMEM_SHARED((NPAD,), jnp.float32),
            pltpu.SemaphoreType.DMA,
        ],
    )
    return k(dst, ones_v, zeros_v)


def _sc_agg_body(D, p_hbm, src_hbm, dst_hbm, zeros_hbm, out_hbm,
                 sidx, didx, rows, zbuf, accum, sg, ss):
    c = lax.axis_index("c")
    s = lax.axis_index("s")

    pltpu.sync_copy(zeros_hbm, zbuf)
    base = s * ROWS_PT
    for r in range(ROWS_PT // ZR):
        pltpu.sync_copy(zbuf, accum.at[pl.ds(base + r * ZR, ZR)])
    plsc.subcore_barrier()

    off = c * NPAD
    nblk = (NBLK - s + (NS - 1)) // NS

    def blk(k, _):
        e0 = (s + k * NS) * BLK
        pltpu.sync_copy(src_hbm.at[pl.ds(e0, BLK)], sidx)
        pltpu.sync_copy(dst_hbm.at[pl.ds(e0, BLK)], didx)

        def adj(q, _):
            row = q // (GP // LANES)
            colq = q % (GP // LANES)
            v = sidx[row, pl.ds(colq * LANES, LANES)]
            sidx[row, pl.ds(colq * LANES, LANES)] = v + off
            return 0
        lax.fori_loop(0, BLK // LANES, adj, 0)

        gd = [
            pltpu.async_copy(p_hbm.at[sidx.at[j]],
                             rows.at[pl.ds(j * GP, GP)], sg)
            for j in range(GPB)
        ]
        for d in gd:
            d.wait()
        sd = [
            pltpu.async_copy(rows.at[pl.ds(j * GP, GP)],
                             accum.at[didx.at[j]], ss, add=True)
            for j in range(GPB)
        ]
        for d in sd:
            d.wait()
        return 0

    lax.fori_loop(0, nblk, blk, 0)
    plsc.subcore_barrier()
    pltpu.sync_copy(accum.at[pl.ds(base, ROWS_PT)],
                    out_hbm.at[c].at[pl.ds(base, ROWS_PT)])


def _sc_edge_agg(p_flat, src, dst, zeros_v, D):
    mesh = plsc.VectorSubcoreMesh(core_axis_name="c", subcore_axis_name="s",
                                  num_cores=NC, num_subcores=NS)
    k = pl.kernel(
        functools.partial(_sc_agg_body, D),
        out_type=jax.ShapeDtypeStruct((NC, NPAD, D), jnp.float32),
        mesh=mesh,
        scratch_types=[
            pltpu.VMEM((GPB, GP), jnp.int32),
            pltpu.VMEM((GPB, GP), jnp.int32),
            pltpu.VMEM((BLK, D), jnp.float32),
            pltpu.VMEM((ZR, D), jnp.float32),
            pltpu.VMEM_SHARED((NPAD, D), jnp.float32),
            pltpu.SemaphoreType.DMA,
            pltpu.SemaphoreType.DMA,
        ],
    )
    return k(p_flat, src, dst, zeros_v)


def _tc_k2_body(x_ref, w_ref, dp_ref, p_ref, dinv_ref):
    dp = dp_ref[...]
    deg = dp[0] + dp[1] + 1.0
    dinv = lax.rsqrt(deg)
    h = jnp.dot(x_ref[...], w_ref[...], preferred_element_type=jnp.float32)
    p = h * dinv
    p_ref[0] = p[:, :32]
    p_ref[1] = p[:, 32:]
    dinv_ref[...] = dinv


def _tc_k2(xp, W1, degp):
    return pl.pallas_call(
        _tc_k2_body,
        grid=(NRB,),
        in_specs=[
            pl.BlockSpec((RB, IN_CH), lambda i: (i, 0)),
            pl.BlockSpec((IN_CH, HID_CH), lambda i: (0, 0)),
            pl.BlockSpec((NC, RB, 1), lambda i: (0, i, 0)),
        ],
        out_specs=[
            pl.BlockSpec((NC, RB, 32), lambda i: (0, i, 0)),
            pl.BlockSpec((RB, 1), lambda i: (i, 0)),
        ],
        out_shape=[
            jax.ShapeDtypeStruct((NC, NPAD, 32), jnp.float32),
            jax.ShapeDtypeStruct((NPAD, 1), jnp.float32),
        ],
    )(xp, W1, degp)


def _tc_k4_body(a_ref, p_ref, d_ref, b1_ref, w2_ref, o_ref):
    di = d_ref[...]
    pre = a_ref[...] + p_ref[...]
    h0 = jnp.maximum(pre[0] * di + b1_ref[0:1, 0:32], 0.0)
    h1 = jnp.maximum(pre[1] * di + b1_ref[0:1, 32:64], 0.0)
    m = (jnp.dot(h0, w2_ref[0:32, :], preferred_element_type=jnp.float32)
         + jnp.dot(h1, w2_ref[32:64, :], preferred_element_type=jnp.float32))
    p2 = m * di
    o_ref[0] = p2[:, :16]
    o_ref[1] = p2[:, 16:]


def _tc_k4(accum1, P1, dinv, b1r, W2):
    return pl.pallas_call(
        _tc_k4_body,
        grid=(NRB,),
        in_specs=[
            pl.BlockSpec((NC, RB, 32), lambda i: (0, i, 0)),
            pl.BlockSpec((NC, RB, 32), lambda i: (0, i, 0)),
            pl.BlockSpec((RB, 1), lambda i: (i, 0)),
            pl.BlockSpec((8, HID_CH), lambda i: (0, 0)),
            pl.BlockSpec((HID_CH, 32), lambda i: (0, 0)),
        ],
        out_specs=pl.BlockSpec((NC, RB, 16), lambda i: (0, i, 0)),
        out_shape=jax.ShapeDtypeStruct((NC, NPAD, 16), jnp.float32),
    )(accum1, P1, dinv, b1r, W2)


def _tc_k6_body(a_ref, p_ref, d_ref, b2_ref, batch_ref, sum_ref, cnt_ref):
    i = pl.program_id(0)
    di = d_ref[...]
    pre = a_ref[...] + p_ref[...]
    h0 = pre[0] * di + b2_ref[0:1, 0:16]
    h1 = pre[1] * di + b2_ref[0:1, 16:32]
    h2 = jnp.concatenate([h0, h1], axis=1)
    bv = batch_ref[0, 0, :]
    seg = lax.broadcasted_iota(jnp.int32, (NUM_GRAPHS, RB), 0)
    oh = (seg == bv[None, :]).astype(jnp.float32)
    part = jnp.dot(oh, h2, preferred_element_type=jnp.float32)
    pcnt = jnp.sum(oh, axis=1, keepdims=True)

    @pl.when(i == 0)
    def _():
        sum_ref[...] = jnp.zeros_like(sum_ref)
        cnt_ref[...] = jnp.zeros_like(cnt_ref)

    sum_ref[...] += part
    cnt_ref[...] += pcnt


def _tc_k6(accum2, P2, dinv, b2r, batch3):
    return pl.pallas_call(
        _tc_k6_body,
        grid=(NRB,),
        in_specs=[
            pl.BlockSpec((NC, RB, 16), lambda i: (0, i, 0)),
            pl.BlockSpec((NC, RB, 16), lambda i: (0, i, 0)),
            pl.BlockSpec((RB, 1), lambda i: (i, 0)),
            pl.BlockSpec((8, 32), lambda i: (0, 0)),
            pl.BlockSpec((1, 1, RB), lambda i: (i, 0, 0)),
        ],
        out_specs=[
            pl.BlockSpec((NUM_GRAPHS, 32), lambda i: (0, 0)),
            pl.BlockSpec((NUM_GRAPHS, 1), lambda i: (0, 0)),
        ],
        out_shape=[
            jax.ShapeDtypeStruct((NUM_GRAPHS, 32), jnp.float32),
            jax.ShapeDtypeStruct((NUM_GRAPHS, 1), jnp.float32),
        ],
    )(accum2, P2, dinv, b2r, batch3)


def _tc_k7_body(s_ref, c_ref, w1_ref, b1_ref, w2_ref, b2_ref, o_ref):
    pooled = s_ref[...] / jnp.maximum(c_ref[...], 1.0)
    t = jnp.maximum(
        jnp.dot(pooled, w1_ref[...], preferred_element_type=jnp.float32)
        + b1_ref[0:1, :], 0.0)
    o_ref[...] = (jnp.dot(t, w2_ref[...], preferred_element_type=jnp.float32)
                  + b2_ref[0:1, :])


def _tc_k7(sums, cnt, fc1_W, fc1_br, fc2_W, fc2_br):
    return pl.pallas_call(
        _tc_k7_body,
        out_shape=jax.ShapeDtypeStruct((NUM_GRAPHS, OUT_CH), jnp.float32),
    )(sums, cnt, fc1_W, fc1_br, fc2_W, fc2_br)


def kernel(x, edge_index, batch, W1, b1, W2, b2, fc1_W, fc1_b, fc2_W, fc2_b):
    src = edge_index[0]
    dst = edge_index[1]

    xp = jnp.pad(x, ((0, NPAD - N), (0, 0)))
    batchp = jnp.pad(batch, (0, NPAD - N),
                     constant_values=NUM_GRAPHS).reshape(NRB, 1, RB)
    ones_v = jnp.ones((GP,), jnp.float32)
    zeros1 = jnp.zeros((ZR,), jnp.float32)
    zeros32 = jnp.zeros((ZR, 32), jnp.float32)
    zeros16 = jnp.zeros((ZR, 16), jnp.float32)
    b1r = jnp.broadcast_to(b1[None, :], (8, HID_CH))
    b2r = jnp.broadcast_to(b2[None, :], (8, OUT_CH))
    fc1_br = jnp.broadcast_to(fc1_b[None, :], (8, OUT_CH))
    fc2_br = jnp.broadcast_to(fc2_b[None, :], (8, OUT_CH))

    degp = _sc_degree(dst, ones_v, zeros1)
    P1, dinv = _tc_k2(xp, W1, degp.reshape(NC, NPAD, 1))
    accum1 = _sc_edge_agg(P1.reshape(NC * NPAD, 32), src, dst, zeros32, 32)
    P2 = _tc_k4(accum1, P1, dinv, b1r, W2)
    accum2 = _sc_edge_agg(P2.reshape(NC * NPAD, 16), src, dst, zeros16, 16)
    sums, cnt = _tc_k6(accum2, P2, dinv, b2r, batchp)
    return _tc_k7(sums, cnt, fc1_W, fc1_br, fc2_W, fc2_br)


# speedup vs baseline: 17.6924x; 17.6924x over previous
import functools

import jax
import jax.numpy as jnp
from jax import lax
from jax.experimental import pallas as pl
from jax.experimental.pallas import tpu as pltpu
from jax.experimental.pallas import tpu_sc as plsc

N = 50000
E = 800000
IN_CH = 128
HID_CH = 64
OUT_CH = 32
NUM_GRAPHS = 256

NC, NS, LANES = 2, 16, 16
NPAD = 50176
ROWS_PT = NPAD // NS
GP = 128
GPB = 8
BLK = GP * GPB
NBLK = 782
EPAD = NBLK * BLK
ZR = 448
SD = 16
RB = 256
NRB = NPAD // RB


def _sc_degree_body(dst_hbm, ones_hbm, zeros_hbm, out_hbm,
                    didx, ones_v, zbuf, hist, sem):
    c = lax.axis_index("c")
    s = lax.axis_index("s")
    wid = c * NS + s

    pltpu.sync_copy(zeros_hbm, zbuf)
    base = s * ROWS_PT
    for r in range(ROWS_PT // ZR):
        pltpu.sync_copy(zbuf, hist.at[pl.ds(base + r * ZR, ZR)])
    pltpu.sync_copy(ones_hbm, ones_v)
    plsc.subcore_barrier()

    nblk = (NBLK - wid + (NC * NS - 1)) // (NC * NS)

    def blk(k, _):
        g0 = (wid + k * NC * NS) * GPB
        pltpu.sync_copy(dst_hbm.at[pl.ds(g0, GPB)], didx)
        descs = [
            pltpu.async_copy(ones_v, hist.at[didx.at[j]], sem, add=True)
            for j in range(GPB)
        ]
        for d in descs:
            d.wait()
        return 0

    lax.fori_loop(0, nblk, blk, 0)
    plsc.subcore_barrier()
    for r in range(ROWS_PT // ZR):
        pltpu.sync_copy(hist.at[pl.ds(base + r * ZR, ZR)], zbuf)
        pltpu.sync_copy(zbuf, out_hbm.at[pl.ds(c * NPAD + base + r * ZR, ZR)])


def _sc_degree(dst, ones_v, zeros_v):
    mesh = plsc.VectorSubcoreMesh(core_axis_name="c", subcore_axis_name="s",
                                  num_cores=NC, num_subcores=NS)
    k = pl.kernel(
        _sc_degree_body,
        out_type=jax.ShapeDtypeStruct((NC * NPAD,), jnp.float32),
        mesh=mesh,
        scratch_types=[
            pltpu.VMEM((GPB, GP), jnp.int32),
            pltpu.VMEM((GP,), jnp.float32),
            pltpu.VMEM((ZR,), jnp.float32),
            pltpu.VMEM_SHARED((NPAD,), jnp.float32),
            pltpu.SemaphoreType.DMA,
        ],
    )
    return k(dst, ones_v, zeros_v)


def _sc_agg_body(SPC, p_hbm, src_hbm, dst_hbm, zeros_hbm, out_hbm,
                 sidx, didx, rows, zbuf, accum, sg, ss):
    c = lax.axis_index("c")
    s = lax.axis_index("s")
    base = s * ROWS_PT
    nblk = (NBLK - s + (NS - 1)) // NS

    for t in range(SPC):
        slab = c * SPC + t
        off = slab * NPAD

        pltpu.sync_copy(zeros_hbm, zbuf)
        for r in range(ROWS_PT // ZR):
            pltpu.sync_copy(zbuf, accum.at[pl.ds(base + r * ZR, ZR)])
        plsc.subcore_barrier()

        def blk(k, _):
            g0 = (s + k * NS) * GPB
            pltpu.sync_copy(src_hbm.at[pl.ds(g0, GPB)], sidx)
            pltpu.sync_copy(dst_hbm.at[pl.ds(g0, GPB)], didx)

            for j in range(GPB):
                def adj(q, _):
                    v = sidx[j, pl.ds(q * LANES, LANES)]
                    sidx[j, pl.ds(q * LANES, LANES)] = v + off
                    return 0
                lax.fori_loop(0, GP // LANES, adj, 0)

            gd = [
                pltpu.async_copy(p_hbm.at[sidx.at[j]],
                                 rows.at[pl.ds(j * GP, GP)], sg)
                for j in range(GPB)
            ]
            for d in gd:
                d.wait()
            sd = [
                pltpu.async_copy(rows.at[pl.ds(j * GP, GP)],
                                 accum.at[didx.at[j]], ss, add=True)
                for j in range(GPB)
            ]
            for d in sd:
                d.wait()
            return 0

        lax.fori_loop(0, nblk, blk, 0)
        plsc.subcore_barrier()
        for r in range(ROWS_PT // ZR):
            pltpu.sync_copy(accum.at[pl.ds(base + r * ZR, ZR)], zbuf)
            pltpu.sync_copy(zbuf, out_hbm.at[slab].at[pl.ds(base + r * ZR, ZR)])
        plsc.subcore_barrier()


def _sc_edge_agg(p_flat, src, dst, zeros_v, nslab):
    mesh = plsc.VectorSubcoreMesh(core_axis_name="c", subcore_axis_name="s",
                                  num_cores=NC, num_subcores=NS)
    k = pl.kernel(
        functools.partial(_sc_agg_body, nslab // NC),
        out_type=jax.ShapeDtypeStruct((nslab, NPAD, SD), jnp.float32),
        mesh=mesh,
        compiler_params=pltpu.CompilerParams(use_tc_tiling_on_sc=False),
        scratch_types=[
            pltpu.VMEM((GPB, GP), jnp.int32),
            pltpu.VMEM((GPB, GP), jnp.int32),
            pltpu.VMEM((BLK, SD), jnp.float32),
            pltpu.VMEM((ZR, SD), jnp.float32),
            pltpu.VMEM_SHARED((NPAD, SD), jnp.float32),
            pltpu.SemaphoreType.DMA,
            pltpu.SemaphoreType.DMA,
        ],
    )
    return k(p_flat, src, dst, zeros_v)


def _tc_k2_body(x_ref, w_ref, dp_ref, p_ref, dinv_ref):
    dp = dp_ref[...]
    deg = dp[0] + dp[1] + 1.0
    dinv = lax.rsqrt(deg)
    h = jnp.dot(x_ref[...], w_ref[...], preferred_element_type=jnp.float32)
    p = h * dinv
    for t in range(HID_CH // SD):
        p_ref[t] = p[:, t * SD:(t + 1) * SD]
    dinv_ref[...] = dinv


def _tc_k2(xp, W1, degp):
    return pl.pallas_call(
        _tc_k2_body,
        grid=(NRB,),
        in_specs=[
            pl.BlockSpec((RB, IN_CH), lambda i: (i, 0)),
            pl.BlockSpec((IN_CH, HID_CH), lambda i: (0, 0)),
            pl.BlockSpec((NC, RB, 1), lambda i: (0, i, 0)),
        ],
        out_specs=[
            pl.BlockSpec((4, RB, SD), lambda i: (0, i, 0)),
            pl.BlockSpec((RB, 1), lambda i: (i, 0)),
        ],
        out_shape=[
            jax.ShapeDtypeStruct((4, NPAD, SD), jnp.float32),
            jax.ShapeDtypeStruct((NPAD, 1), jnp.float32),
        ],
    )(xp, W1, degp)


def _tc_k4_body(a_ref, p_ref, d_ref, b1_ref, w2_ref, o_ref):
    di = d_ref[...]
    pre = a_ref[...] + p_ref[...]
    m = jnp.zeros((RB, OUT_CH), jnp.float32)
    for t in range(HID_CH // SD):
        ht = jnp.maximum(pre[t] * di + b1_ref[0:1, t * SD:(t + 1) * SD], 0.0)
        m = m + jnp.dot(ht, w2_ref[t * SD:(t + 1) * SD, :],
                        preferred_element_type=jnp.float32)
    p2 = m * di
    o_ref[0] = p2[:, :16]
    o_ref[1] = p2[:, 16:]


def _tc_k4(accum1, P1, dinv, b1r, W2):
    return pl.pallas_call(
        _tc_k4_body,
        grid=(NRB,),
        in_specs=[
            pl.BlockSpec((4, RB, SD), lambda i: (0, i, 0)),
            pl.BlockSpec((4, RB, SD), lambda i: (0, i, 0)),
            pl.BlockSpec((RB, 1), lambda i: (i, 0)),
            pl.BlockSpec((8, HID_CH), lambda i: (0, 0)),
            pl.BlockSpec((HID_CH, 32), lambda i: (0, 0)),
        ],
        out_specs=pl.BlockSpec((NC, RB, 16), lambda i: (0, i, 0)),
        out_shape=jax.ShapeDtypeStruct((NC, NPAD, 16), jnp.float32),
    )(accum1, P1, dinv, b1r, W2)


def _tc_k6_body(a_ref, p_ref, d_ref, b2_ref, batch_ref, sum_ref, cnt_ref):
    i = pl.program_id(0)
    di = d_ref[...]
    pre = a_ref[...] + p_ref[...]
    h0 = pre[0] * di + b2_ref[0:1, 0:16]
    h1 = pre[1] * di + b2_ref[0:1, 16:32]
    h2 = jnp.concatenate([h0, h1], axis=1)
    bv = batch_ref[0, 0, :]
    seg = lax.broadcasted_iota(jnp.int32, (NUM_GRAPHS, RB), 0)
    oh = (seg == bv[None, :]).astype(jnp.float32)
    part = jnp.dot(oh, h2, preferred_element_type=jnp.float32)
    pcnt = jnp.sum(oh, axis=1, keepdims=True)

    @pl.when(i == 0)
    def _():
        sum_ref[...] = jnp.zeros_like(sum_ref)
        cnt_ref[...] = jnp.zeros_like(cnt_ref)

    sum_ref[...] += part
    cnt_ref[...] += pcnt


def _tc_k6(accum2, P2, dinv, b2r, batch3):
    return pl.pallas_call(
        _tc_k6_body,
        grid=(NRB,),
        in_specs=[
            pl.BlockSpec((NC, RB, 16), lambda i: (0, i, 0)),
            pl.BlockSpec((NC, RB, 16), lambda i: (0, i, 0)),
            pl.BlockSpec((RB, 1), lambda i: (i, 0)),
            pl.BlockSpec((8, 32), lambda i: (0, 0)),
            pl.BlockSpec((1, 1, RB), lambda i: (i, 0, 0)),
        ],
        out_specs=[
            pl.BlockSpec((NUM_GRAPHS, 32), lambda i: (0, 0)),
            pl.BlockSpec((NUM_GRAPHS, 1), lambda i: (0, 0)),
        ],
        out_shape=[
            jax.ShapeDtypeStruct((NUM_GRAPHS, 32), jnp.float32),
            jax.ShapeDtypeStruct((NUM_GRAPHS, 1), jnp.float32),
        ],
    )(accum2, P2, dinv, b2r, batch3)


def _tc_k7_body(s_ref, c_ref, w1_ref, b1_ref, w2_ref, b2_ref, o_ref):
    pooled = s_ref[...] / jnp.maximum(c_ref[...], 1.0)
    t = jnp.maximum(
        jnp.dot(pooled, w1_ref[...], preferred_element_type=jnp.float32)
        + b1_ref[0:1, :], 0.0)
    o_ref[...] = (jnp.dot(t, w2_ref[...], preferred_element_type=jnp.float32)
                  + b2_ref[0:1, :])


def _tc_k7(sums, cnt, fc1_W, fc1_br, fc2_W, fc2_br):
    return pl.pallas_call(
        _tc_k7_body,
        out_shape=jax.ShapeDtypeStruct((NUM_GRAPHS, OUT_CH), jnp.float32),
    )(sums, cnt, fc1_W, fc1_br, fc2_W, fc2_br)


def kernel(x, edge_index, batch, W1, b1, W2, b2, fc1_W, fc1_b, fc2_W, fc2_b):
    src = jnp.pad(edge_index[0], (0, EPAD - E)).reshape(NBLK * GPB, GP)
    dst = jnp.pad(edge_index[1], (0, EPAD - E),
                  constant_values=N).reshape(NBLK * GPB, GP)

    xp = jnp.pad(x, ((0, NPAD - N), (0, 0)))
    batchp = jnp.pad(batch, (0, NPAD - N),
                     constant_values=NUM_GRAPHS).reshape(NRB, 1, RB)
    ones_v = jnp.ones((GP,), jnp.float32)
    zeros1 = jnp.zeros((ZR,), jnp.float32)
    zeros16 = jnp.zeros((ZR, SD), jnp.float32)
    b1r = jnp.broadcast_to(b1[None, :], (8, HID_CH))
    b2r = jnp.broadcast_to(b2[None, :], (8, OUT_CH))
    fc1_br = jnp.broadcast_to(fc1_b[None, :], (8, OUT_CH))
    fc2_br = jnp.broadcast_to(fc2_b[None, :], (8, OUT_CH))

    degp = _sc_degree(dst, ones_v, zeros1)
    P1, dinv = _tc_k2(xp, W1, degp.reshape(NC, NPAD, 1))
    accum1 = _sc_edge_agg(P1.reshape(4 * NPAD, SD), src, dst, zeros16, 4)
    P2 = _tc_k4(accum1, P1, dinv, b1r, W2)
    accum2 = _sc_edge_agg(P2.reshape(NC * NPAD, SD), src, dst, zeros16, 2)
    sums, cnt = _tc_k6(accum2, P2, dinv, b2r, batchp)
    return _tc_k7(sums, cnt, fc1_W, fc1_br, fc2_W, fc2_br)


# speedup vs baseline: 29.9897x; 1.6951x over previous
import functools

import jax
import jax.numpy as jnp
from jax import lax
from jax.experimental import pallas as pl
from jax.experimental.pallas import tpu as pltpu
from jax.experimental.pallas import tpu_sc as plsc

N = 50000
E = 800000
IN_CH = 128
HID_CH = 64
OUT_CH = 32
NUM_GRAPHS = 256

NC, NS, LANES = 2, 16, 16
NPAD = 50176
ROWS_PT = NPAD // NS
GP = 128
GPB = 8
BLK = GP * GPB
NBLK = 782
EPAD = NBLK * BLK
ZR = 448
SD = 16
RB = 1024
NRB = NPAD // RB


def _sc_degree_body(dst_hbm, ones_hbm, zeros_hbm, out_hbm,
                    didx, ones_v, zbuf, hist, sem):
    c = lax.axis_index("c")
    s = lax.axis_index("s")
    wid = c * NS + s

    pltpu.sync_copy(zeros_hbm, zbuf)
    base = s * ROWS_PT
    for r in range(ROWS_PT // ZR):
        pltpu.sync_copy(zbuf, hist.at[pl.ds(base + r * ZR, ZR)])
    pltpu.sync_copy(ones_hbm, ones_v)
    plsc.subcore_barrier()

    nblk = (NBLK - wid + (NC * NS - 1)) // (NC * NS)

    def blk(k, _):
        g0 = (wid + k * NC * NS) * GPB
        pltpu.sync_copy(dst_hbm.at[pl.ds(g0, GPB)], didx)
        descs = [
            pltpu.async_copy(ones_v, hist.at[didx.at[j]], sem, add=True)
            for j in range(GPB)
        ]
        for d in descs:
            d.wait()
        return 0

    lax.fori_loop(0, nblk, blk, 0)
    plsc.subcore_barrier()
    for r in range(ROWS_PT // ZR):
        pltpu.sync_copy(hist.at[pl.ds(base + r * ZR, ZR)], zbuf)
        pltpu.sync_copy(zbuf, out_hbm.at[pl.ds(c * NPAD + base + r * ZR, ZR)])


def _sc_degree(dst, ones_v, zeros_v):
    mesh = plsc.VectorSubcoreMesh(core_axis_name="c", subcore_axis_name="s",
                                  num_cores=NC, num_subcores=NS)
    k = pl.kernel(
        _sc_degree_body,
        out_type=jax.ShapeDtypeStruct((NC * NPAD,), jnp.float32),
        mesh=mesh,
        scratch_types=[
            pltpu.VMEM((GPB, GP), jnp.int32),
            pltpu.VMEM((GP,), jnp.float32),
            pltpu.VMEM((ZR,), jnp.float32),
            pltpu.VMEM_SHARED((NPAD,), jnp.float32),
            pltpu.SemaphoreType.DMA,
        ],
    )
    return k(dst, ones_v, zeros_v)


def _sc_agg_body(SPC, p_hbm, src_hbm, dst_hbm, zeros_hbm, out_hbm,
                 sidx, didx, rows, zbuf, accum, sg, ss):
    c = lax.axis_index("c")
    s = lax.axis_index("s")
    base = s * ROWS_PT
    nblk = (NBLK - s + (NS - 1)) // NS

    for t in range(SPC):
        slab = c * SPC + t

        pltpu.sync_copy(zeros_hbm, zbuf)
        for r in range(ROWS_PT // ZR):
            pltpu.sync_copy(zbuf, accum.at[pl.ds(base + r * ZR, ZR)])
        plsc.subcore_barrier()

        def blk(k, _):
            g0 = (s + k * NS) * GPB
            pltpu.sync_copy(src_hbm.at[pl.ds(g0, GPB)], sidx)
            pltpu.sync_copy(dst_hbm.at[pl.ds(g0, GPB)], didx)

            for j in range(GPB):
                def adj(q, _):
                    v = sidx[j, pl.ds(q * LANES, LANES)]
                    sidx[j, pl.ds(q * LANES, LANES)] = v * 8 + slab
                    return 0
                lax.fori_loop(0, GP // LANES, adj, 0)

            gd = [
                pltpu.async_copy(p_hbm.at[sidx.at[j]],
                                 rows.at[pl.ds(j * GP, GP)], sg)
                for j in range(GPB)
            ]
            for d in gd:
                d.wait()
            sd = [
                pltpu.async_copy(rows.at[pl.ds(j * GP, GP)],
                                 accum.at[didx.at[j]], ss, add=True)
                for j in range(GPB)
            ]
            for d in sd:
                d.wait()
            return 0

        lax.fori_loop(0, nblk, blk, 0)
        plsc.subcore_barrier()
        for r in range(ROWS_PT // ZR):
            pltpu.sync_copy(accum.at[pl.ds(base + r * ZR, ZR)], zbuf)
            pltpu.sync_copy(zbuf, out_hbm.at[pl.ds(base + r * ZR, ZR),
                                             pl.ds(slab * SD, SD)])
        plsc.subcore_barrier()


def _sc_edge_agg(p_flat, src, dst, zeros_v, nslab):
    mesh = plsc.VectorSubcoreMesh(core_axis_name="c", subcore_axis_name="s",
                                  num_cores=NC, num_subcores=NS)
    k = pl.kernel(
        functools.partial(_sc_agg_body, nslab // NC),
        out_type=jax.ShapeDtypeStruct((NPAD, 128), jnp.float32),
        mesh=mesh,
        compiler_params=pltpu.CompilerParams(use_tc_tiling_on_sc=False),
        scratch_types=[
            pltpu.VMEM((GPB, GP), jnp.int32),
            pltpu.VMEM((GPB, GP), jnp.int32),
            pltpu.VMEM((BLK, SD), jnp.float32),
            pltpu.VMEM((ZR, SD), jnp.float32),
            pltpu.VMEM_SHARED((NPAD, SD), jnp.float32),
            pltpu.SemaphoreType.DMA,
            pltpu.SemaphoreType.DMA,
        ],
    )
    return k(p_flat, src, dst, zeros_v)


def _dcol_from_degp(dp):
    deg = dp[0] + dp[1] + 1.0
    dinv = lax.rsqrt(deg)
    ident = (lax.broadcasted_iota(jnp.int32, (128, 128), 0) ==
             lax.broadcasted_iota(jnp.int32, (128, 128), 1)
             ).astype(jnp.float32)
    cols = [lax.dot_general(ident, dinv[r:r + 1, :],
                            (((1,), (1,)), ((), ())),
                            preferred_element_type=jnp.float32)
            for r in range(RB // 128)]
    return jnp.concatenate(cols, axis=0)


def _tc_k2_body(x_ref, w_ref, dp_ref, p_ref):
    dcol = _dcol_from_degp(dp_ref[...])
    h = jnp.dot(x_ref[...], w_ref[...], preferred_element_type=jnp.float32)
    p = h * dcol
    p_ref[...] = jnp.concatenate(
        [p, jnp.zeros((RB, 128 - HID_CH), jnp.float32)], axis=1)


def _tc_k2(xp, W1, degp):
    return pl.pallas_call(
        _tc_k2_body,
        grid=(NRB,),
        in_specs=[
            pl.BlockSpec((RB, IN_CH), lambda i: (i, 0)),
            pl.BlockSpec((IN_CH, HID_CH), lambda i: (0, 0)),
            pl.BlockSpec((NC, RB // 128, 128), lambda i: (0, i, 0)),
        ],
        out_specs=pl.BlockSpec((RB, 128), lambda i: (i, 0)),
        out_shape=jax.ShapeDtypeStruct((NPAD, 128), jnp.float32),
    )(xp, W1, degp)


def _tc_k4_body(a_ref, p_ref, d_ref, b1_ref, w2_ref, o_ref):
    dcol = _dcol_from_degp(d_ref[...])
    acc = a_ref[...][:, :HID_CH]
    p1 = p_ref[...][:, :HID_CH]
    h1 = jnp.maximum((acc + p1) * dcol + b1_ref[0:1, :], 0.0)
    m = jnp.dot(h1, w2_ref[...], preferred_element_type=jnp.float32)
    p2 = m * dcol
    o_ref[...] = jnp.concatenate(
        [p2, jnp.zeros((RB, 128 - OUT_CH), jnp.float32)], axis=1)


def _tc_k4(accum1, P1, dinv, b1r, W2):
    return pl.pallas_call(
        _tc_k4_body,
        grid=(NRB,),
        in_specs=[
            pl.BlockSpec((RB, 128), lambda i: (i, 0)),
            pl.BlockSpec((RB, 128), lambda i: (i, 0)),
            pl.BlockSpec((NC, RB // 128, 128), lambda i: (0, i, 0)),
            pl.BlockSpec((8, HID_CH), lambda i: (0, 0)),
            pl.BlockSpec((HID_CH, 32), lambda i: (0, 0)),
        ],
        out_specs=pl.BlockSpec((RB, 128), lambda i: (i, 0)),
        out_shape=jax.ShapeDtypeStruct((NPAD, 128), jnp.float32),
    )(accum1, P1, dinv, b1r, W2)


def _tc_k6_body(a_ref, p_ref, d_ref, b2_ref, batch_ref, sum_ref, cnt_ref):
    i = pl.program_id(0)
    dcol = _dcol_from_degp(d_ref[...])
    acc = a_ref[...][:, :OUT_CH]
    p2 = p_ref[...][:, :OUT_CH]
    h2 = (acc + p2) * dcol + b2_ref[0:1, :]
    bv = batch_ref[0, 0, :]
    seg = lax.broadcasted_iota(jnp.int32, (NUM_GRAPHS, RB), 0)
    oh = (seg == bv[None, :]).astype(jnp.float32)
    part = jnp.dot(oh, h2, preferred_element_type=jnp.float32)
    pcnt = jnp.sum(oh, axis=1, keepdims=True)

    @pl.when(i == 0)
    def _():
        sum_ref[...] = jnp.zeros_like(sum_ref)
        cnt_ref[...] = jnp.zeros_like(cnt_ref)

    sum_ref[...] += part
    cnt_ref[...] += pcnt


def _tc_k6(accum2, P2, dinv, b2r, batch3):
    return pl.pallas_call(
        _tc_k6_body,
        grid=(NRB,),
        in_specs=[
            pl.BlockSpec((RB, 128), lambda i: (i, 0)),
            pl.BlockSpec((RB, 128), lambda i: (i, 0)),
            pl.BlockSpec((NC, RB // 128, 128), lambda i: (0, i, 0)),
            pl.BlockSpec((8, 32), lambda i: (0, 0)),
            pl.BlockSpec((1, 1, RB), lambda i: (i, 0, 0)),
        ],
        out_specs=[
            pl.BlockSpec((NUM_GRAPHS, 32), lambda i: (0, 0)),
            pl.BlockSpec((NUM_GRAPHS, 1), lambda i: (0, 0)),
        ],
        out_shape=[
            jax.ShapeDtypeStruct((NUM_GRAPHS, 32), jnp.float32),
            jax.ShapeDtypeStruct((NUM_GRAPHS, 1), jnp.float32),
        ],
    )(accum2, P2, dinv, b2r, batch3)


def _tc_k7_body(s_ref, c_ref, w1_ref, b1_ref, w2_ref, b2_ref, o_ref):
    pooled = s_ref[...] / jnp.maximum(c_ref[...], 1.0)
    t = jnp.maximum(
        jnp.dot(pooled, w1_ref[...], preferred_element_type=jnp.float32)
        + b1_ref[0:1, :], 0.0)
    o_ref[...] = (jnp.dot(t, w2_ref[...], preferred_element_type=jnp.float32)
                  + b2_ref[0:1, :])


def _tc_k7(sums, cnt, fc1_W, fc1_br, fc2_W, fc2_br):
    return pl.pallas_call(
        _tc_k7_body,
        out_shape=jax.ShapeDtypeStruct((NUM_GRAPHS, OUT_CH), jnp.float32),
    )(sums, cnt, fc1_W, fc1_br, fc2_W, fc2_br)


def kernel(x, edge_index, batch, W1, b1, W2, b2, fc1_W, fc1_b, fc2_W, fc2_b):
    src = jnp.pad(edge_index[0], (0, EPAD - E)).reshape(NBLK * GPB, GP)
    dst = jnp.pad(edge_index[1], (0, EPAD - E),
                  constant_values=N).reshape(NBLK * GPB, GP)

    xp = jnp.pad(x, ((0, NPAD - N), (0, 0)))
    batchp = jnp.pad(batch, (0, NPAD - N),
                     constant_values=NUM_GRAPHS).reshape(NRB, 1, RB)
    ones_v = jnp.ones((GP,), jnp.float32)
    zeros1 = jnp.zeros((ZR,), jnp.float32)
    zeros16 = jnp.zeros((ZR, SD), jnp.float32)
    b1r = jnp.broadcast_to(b1[None, :], (8, HID_CH))
    b2r = jnp.broadcast_to(b2[None, :], (8, OUT_CH))
    fc1_br = jnp.broadcast_to(fc1_b[None, :], (8, OUT_CH))
    fc2_br = jnp.broadcast_to(fc2_b[None, :], (8, OUT_CH))

    degp = _sc_degree(dst, ones_v, zeros1).reshape(NC, NPAD // 128, 128)
    P1 = _tc_k2(xp, W1, degp)
    accum1 = _sc_edge_agg(P1.reshape(8 * NPAD, SD), src, dst, zeros16, 4)
    P2 = _tc_k4(accum1, P1, degp, b1r, W2)
    accum2 = _sc_edge_agg(P2.reshape(8 * NPAD, SD), src, dst, zeros16, 2)
    sums, cnt = _tc_k6(accum2, P2, degp, b2r, batchp)
    return _tc_k7(sums, cnt, fc1_W, fc1_br, fc2_W, fc2_br)


# speedup vs baseline: 34.7040x; 1.1572x over previous
import functools

import jax
import jax.numpy as jnp
from jax import lax
from jax.experimental import pallas as pl
from jax.experimental.pallas import tpu as pltpu
from jax.experimental.pallas import tpu_sc as plsc

N = 50000
E = 800000
IN_CH = 128
HID_CH = 64
OUT_CH = 32
NUM_GRAPHS = 256

NC, NS, LANES = 2, 16, 16
NPAD = 50176
ROWS_PT = NPAD // NS
GP = 128
GPB = 16
BLK = GP * GPB
NBLK = 391
EPAD = NBLK * BLK
ZR = 448
SD = 16
RB = 1024
NRB = NPAD // RB


def _sc_degree_body(dst_hbm, ones_hbm, zeros_hbm, out_hbm,
                    didx, ones_v, zbuf, hist, sem):
    c = lax.axis_index("c")
    s = lax.axis_index("s")
    wid = c * NS + s

    pltpu.sync_copy(zeros_hbm, zbuf)
    base = s * ROWS_PT
    for r in range(ROWS_PT // ZR):
        pltpu.sync_copy(zbuf, hist.at[pl.ds(base + r * ZR, ZR)])
    pltpu.sync_copy(ones_hbm, ones_v)
    plsc.subcore_barrier()

    nblk = (NBLK - wid + (NC * NS - 1)) // (NC * NS)

    def blk(k, _):
        g0 = (wid + k * NC * NS) * GPB
        pltpu.sync_copy(dst_hbm.at[pl.ds(g0, GPB)], didx)
        descs = [
            pltpu.async_copy(ones_v, hist.at[didx.at[j]], sem, add=True)
            for j in range(GPB)
        ]
        for d in descs:
            d.wait()
        return 0

    lax.fori_loop(0, nblk, blk, 0)
    plsc.subcore_barrier()
    for r in range(ROWS_PT // ZR):
        pltpu.sync_copy(hist.at[pl.ds(base + r * ZR, ZR)], zbuf)
        pltpu.sync_copy(zbuf, out_hbm.at[pl.ds(c * NPAD + base + r * ZR, ZR)])


def _sc_degree(dst, ones_v, zeros_v):
    mesh = plsc.VectorSubcoreMesh(core_axis_name="c", subcore_axis_name="s",
                                  num_cores=NC, num_subcores=NS)
    k = pl.kernel(
        _sc_degree_body,
        out_type=jax.ShapeDtypeStruct((NC * NPAD,), jnp.float32),
        mesh=mesh,
        scratch_types=[
            pltpu.VMEM((GPB, GP), jnp.int32),
            pltpu.VMEM((GP,), jnp.float32),
            pltpu.VMEM((ZR,), jnp.float32),
            pltpu.VMEM_SHARED((NPAD,), jnp.float32),
            pltpu.SemaphoreType.DMA,
        ],
    )
    return k(dst, ones_v, zeros_v)


def _sc_agg_body(SPC, p_hbm, src_hbm, dst_hbm, zeros_hbm, out_hbm,
                 sidx, didx, rows, zbuf, accum, sg, ss):
    c = lax.axis_index("c")
    s = lax.axis_index("s")
    base = s * ROWS_PT
    nblk = (NBLK - s + (NS - 1)) // NS

    for t in range(SPC):
        slab = c * SPC + t

        pltpu.sync_copy(zeros_hbm, zbuf)
        for r in range(ROWS_PT // ZR):
            pltpu.sync_copy(zbuf, accum.at[pl.ds(base + r * ZR, ZR)])
        plsc.subcore_barrier()

        def blk(k, _):
            g0 = (s + k * NS) * GPB
            pltpu.sync_copy(src_hbm.at[pl.ds(g0, GPB)], sidx)
            pltpu.sync_copy(dst_hbm.at[pl.ds(g0, GPB)], didx)

            for j in range(GPB):
                def adj(q, _):
                    v = sidx[j, pl.ds(q * LANES, LANES)]
                    sidx[j, pl.ds(q * LANES, LANES)] = v * 8 + slab
                    return 0
                lax.fori_loop(0, GP // LANES, adj, 0)

            gd = [
                pltpu.async_copy(p_hbm.at[sidx.at[j]],
                                 rows.at[pl.ds(j * GP, GP)], sg)
                for j in range(GPB)
            ]
            for d in gd:
                d.wait()
            sd = [
                pltpu.async_copy(rows.at[pl.ds(j * GP, GP)],
                                 accum.at[didx.at[j]], ss, add=True)
                for j in range(GPB)
            ]
            for d in sd:
                d.wait()
            return 0

        lax.fori_loop(0, nblk, blk, 0)
        plsc.subcore_barrier()
        for r in range(ROWS_PT // ZR):
            pltpu.sync_copy(accum.at[pl.ds(base + r * ZR, ZR)], zbuf)
            pltpu.sync_copy(zbuf, out_hbm.at[pl.ds(base + r * ZR, ZR),
                                             pl.ds(slab * SD, SD)])
        plsc.subcore_barrier()


def _sc_edge_agg(p_flat, src, dst, zeros_v, nslab):
    mesh = plsc.VectorSubcoreMesh(core_axis_name="c", subcore_axis_name="s",
                                  num_cores=NC, num_subcores=NS)
    k = pl.kernel(
        functools.partial(_sc_agg_body, nslab // NC),
        out_type=jax.ShapeDtypeStruct((NPAD, 128), jnp.float32),
        mesh=mesh,
        compiler_params=pltpu.CompilerParams(use_tc_tiling_on_sc=False),
        scratch_types=[
            pltpu.VMEM((GPB, GP), jnp.int32),
            pltpu.VMEM((GPB, GP), jnp.int32),
            pltpu.VMEM((BLK, SD), jnp.float32),
            pltpu.VMEM((ZR, SD), jnp.float32),
            pltpu.VMEM_SHARED((NPAD, SD), jnp.float32),
            pltpu.SemaphoreType.DMA,
            pltpu.SemaphoreType.DMA,
        ],
    )
    return k(p_flat, src, dst, zeros_v)


def _dcol_from_degp(dp):
    deg = dp[0] + dp[1] + 1.0
    dinv = lax.rsqrt(deg)
    ident = (lax.broadcasted_iota(jnp.int32, (128, 128), 0) ==
             lax.broadcasted_iota(jnp.int32, (128, 128), 1)
             ).astype(jnp.float32)
    cols = [lax.dot_general(ident, dinv[r:r + 1, :],
                            (((1,), (1,)), ((), ())),
                            preferred_element_type=jnp.float32)
            for r in range(RB // 128)]
    return jnp.concatenate(cols, axis=0)


def _tc_k2_body(x_ref, w_ref, dp_ref, p_ref):
    dcol = _dcol_from_degp(dp_ref[...])
    h = jnp.dot(x_ref[...], w_ref[...], preferred_element_type=jnp.float32)
    p = h * dcol
    p_ref[...] = jnp.concatenate(
        [p, jnp.zeros((RB, 128 - HID_CH), jnp.float32)], axis=1)


def _tc_k2(xp, W1, degp):
    return pl.pallas_call(
        _tc_k2_body,
        grid=(NRB,),
        in_specs=[
            pl.BlockSpec((RB, IN_CH), lambda i: (i, 0)),
            pl.BlockSpec((IN_CH, HID_CH), lambda i: (0, 0)),
            pl.BlockSpec((NC, RB // 128, 128), lambda i: (0, i, 0)),
        ],
        out_specs=pl.BlockSpec((RB, 128), lambda i: (i, 0)),
        out_shape=jax.ShapeDtypeStruct((NPAD, 128), jnp.float32),
    )(xp, W1, degp)


def _tc_k4_body(a_ref, p_ref, d_ref, b1_ref, w2_ref, o_ref):
    dcol = _dcol_from_degp(d_ref[...])
    acc = a_ref[...][:, :HID_CH]
    p1 = p_ref[...][:, :HID_CH]
    h1 = jnp.maximum((acc + p1) * dcol + b1_ref[0:1, :], 0.0)
    m = jnp.dot(h1, w2_ref[...], preferred_element_type=jnp.float32)
    p2 = m * dcol
    o_ref[...] = jnp.concatenate(
        [p2, jnp.zeros((RB, 128 - OUT_CH), jnp.float32)], axis=1)


def _tc_k4(accum1, P1, dinv, b1r, W2):
    return pl.pallas_call(
        _tc_k4_body,
        grid=(NRB,),
        in_specs=[
            pl.BlockSpec((RB, 128), lambda i: (i, 0)),
            pl.BlockSpec((RB, 128), lambda i: (i, 0)),
            pl.BlockSpec((NC, RB // 128, 128), lambda i: (0, i, 0)),
            pl.BlockSpec((8, HID_CH), lambda i: (0, 0)),
            pl.BlockSpec((HID_CH, 32), lambda i: (0, 0)),
        ],
        out_specs=pl.BlockSpec((RB, 128), lambda i: (i, 0)),
        out_shape=jax.ShapeDtypeStruct((NPAD, 128), jnp.float32),
    )(accum1, P1, dinv, b1r, W2)


def _tc_k6_body(a_ref, p_ref, d_ref, b2_ref, batch_ref, sum_ref, cnt_ref):
    i = pl.program_id(0)
    dcol = _dcol_from_degp(d_ref[...])
    acc = a_ref[...][:, :OUT_CH]
    p2 = p_ref[...][:, :OUT_CH]
    h2 = (acc + p2) * dcol + b2_ref[0:1, :]
    bv = batch_ref[0, 0, :]
    seg = lax.broadcasted_iota(jnp.int32, (NUM_GRAPHS, RB), 0)
    oh = (seg == bv[None, :]).astype(jnp.float32)
    part = jnp.dot(oh, h2, preferred_element_type=jnp.float32)
    pcnt = jnp.sum(oh, axis=1, keepdims=True)

    @pl.when(i == 0)
    def _():
        sum_ref[...] = jnp.zeros_like(sum_ref)
        cnt_ref[...] = jnp.zeros_like(cnt_ref)

    sum_ref[...] += part
    cnt_ref[...] += pcnt


def _tc_k6(accum2, P2, dinv, b2r, batch3):
    return pl.pallas_call(
        _tc_k6_body,
        grid=(NRB,),
        in_specs=[
            pl.BlockSpec((RB, 128), lambda i: (i, 0)),
            pl.BlockSpec((RB, 128), lambda i: (i, 0)),
            pl.BlockSpec((NC, RB // 128, 128), lambda i: (0, i, 0)),
            pl.BlockSpec((8, 32), lambda i: (0, 0)),
            pl.BlockSpec((1, 1, RB), lambda i: (i, 0, 0)),
        ],
        out_specs=[
            pl.BlockSpec((NUM_GRAPHS, 32), lambda i: (0, 0)),
            pl.BlockSpec((NUM_GRAPHS, 1), lambda i: (0, 0)),
        ],
        out_shape=[
            jax.ShapeDtypeStruct((NUM_GRAPHS, 32), jnp.float32),
            jax.ShapeDtypeStruct((NUM_GRAPHS, 1), jnp.float32),
        ],
    )(accum2, P2, dinv, b2r, batch3)


def _tc_k7_body(s_ref, c_ref, w1_ref, b1_ref, w2_ref, b2_ref, o_ref):
    pooled = s_ref[...] / jnp.maximum(c_ref[...], 1.0)
    t = jnp.maximum(
        jnp.dot(pooled, w1_ref[...], preferred_element_type=jnp.float32)
        + b1_ref[0:1, :], 0.0)
    o_ref[...] = (jnp.dot(t, w2_ref[...], preferred_element_type=jnp.float32)
                  + b2_ref[0:1, :])


def _tc_k7(sums, cnt, fc1_W, fc1_br, fc2_W, fc2_br):
    return pl.pallas_call(
        _tc_k7_body,
        out_shape=jax.ShapeDtypeStruct((NUM_GRAPHS, OUT_CH), jnp.float32),
    )(sums, cnt, fc1_W, fc1_br, fc2_W, fc2_br)


def kernel(x, edge_index, batch, W1, b1, W2, b2, fc1_W, fc1_b, fc2_W, fc2_b):
    src = jnp.pad(edge_index[0], (0, EPAD - E)).reshape(NBLK * GPB, GP)
    dst = jnp.pad(edge_index[1], (0, EPAD - E),
                  constant_values=N).reshape(NBLK * GPB, GP)

    xp = jnp.pad(x, ((0, NPAD - N), (0, 0)))
    batchp = jnp.pad(batch, (0, NPAD - N),
                     constant_values=NUM_GRAPHS).reshape(NRB, 1, RB)
    ones_v = jnp.ones((GP,), jnp.float32)
    zeros1 = jnp.zeros((ZR,), jnp.float32)
    zeros16 = jnp.zeros((ZR, SD), jnp.float32)
    b1r = jnp.broadcast_to(b1[None, :], (8, HID_CH))
    b2r = jnp.broadcast_to(b2[None, :], (8, OUT_CH))
    fc1_br = jnp.broadcast_to(fc1_b[None, :], (8, OUT_CH))
    fc2_br = jnp.broadcast_to(fc2_b[None, :], (8, OUT_CH))

    degp = _sc_degree(dst, ones_v, zeros1).reshape(NC, NPAD // 128, 128)
    P1 = _tc_k2(xp, W1, degp)
    accum1 = _sc_edge_agg(P1.reshape(8 * NPAD, SD), src, dst, zeros16, 4)
    P2 = _tc_k4(accum1, P1, degp, b1r, W2)
    accum2 = _sc_edge_agg(P2.reshape(8 * NPAD, SD), src, dst, zeros16, 2)
    sums, cnt = _tc_k6(accum2, P2, degp, b2r, batchp)
    return _tc_k7(sums, cnt, fc1_W, fc1_br, fc2_W, fc2_br)


# speedup vs baseline: 48.2092x; 1.3892x over previous
import functools

import jax
import jax.numpy as jnp
from jax import lax
from jax.experimental import pallas as pl
from jax.experimental.pallas import tpu as pltpu
from jax.experimental.pallas import tpu_sc as plsc

N = 50000
E = 800000
IN_CH = 128
HID_CH = 64
OUT_CH = 32
NUM_GRAPHS = 256

NC, NS, LANES = 2, 16, 16
NPAD = 50176
ROWS_PT = NPAD // NS
GP = 128
GPB = 16
BLK = GP * GPB
NBLK = 391
EPAD = NBLK * BLK
ZR = 448
SD = 16
RB = 1024
NRB = NPAD // RB


def _sc_degree_body(dst_hbm, ones_hbm, zeros_hbm, out_hbm,
                    didx, ones_v, zbuf, hist, sem):
    c = lax.axis_index("c")
    s = lax.axis_index("s")
    wid = c * NS + s

    pltpu.sync_copy(zeros_hbm, zbuf)
    base = s * ROWS_PT
    for r in range(ROWS_PT // ZR):
        pltpu.sync_copy(zbuf, hist.at[pl.ds(base + r * ZR, ZR)])
    pltpu.sync_copy(ones_hbm, ones_v)
    plsc.subcore_barrier()

    nblk = (NBLK - wid + (NC * NS - 1)) // (NC * NS)

    def blk(k, _):
        g0 = (wid + k * NC * NS) * GPB
        pltpu.sync_copy(dst_hbm.at[pl.ds(g0, GPB)], didx)
        descs = [
            pltpu.async_copy(ones_v, hist.at[didx.at[j]], sem, add=True)
            for j in range(GPB)
        ]
        for d in descs:
            d.wait()
        return 0

    lax.fori_loop(0, nblk, blk, 0)
    plsc.subcore_barrier()
    for r in range(ROWS_PT // ZR):
        pltpu.sync_copy(hist.at[pl.ds(base + r * ZR, ZR)], zbuf)
        pltpu.sync_copy(zbuf, out_hbm.at[pl.ds(c * NPAD + base + r * ZR, ZR)])


def _sc_degree(dst, ones_v, zeros_v):
    mesh = plsc.VectorSubcoreMesh(core_axis_name="c", subcore_axis_name="s",
                                  num_cores=NC, num_subcores=NS)
    k = pl.kernel(
        _sc_degree_body,
        out_type=jax.ShapeDtypeStruct((NC * NPAD,), jnp.float32),
        mesh=mesh,
        scratch_types=[
            pltpu.VMEM((GPB, GP), jnp.int32),
            pltpu.VMEM((GP,), jnp.float32),
            pltpu.VMEM((ZR,), jnp.float32),
            pltpu.VMEM_SHARED((NPAD,), jnp.float32),
            pltpu.SemaphoreType.DMA,
        ],
    )
    return k(dst, ones_v, zeros_v)


def _sc_agg_body(SPC, p_hbm, src_hbm, dst_hbm, zeros_hbm, out_hbm,
                 sidx0, didx0, sidx1, didx1, rows0, rows1, zbuf, accum,
                 sg0, sg1, ss0, ss1, si0, si1):
    c = lax.axis_index("c")
    s = lax.axis_index("s")
    base = s * ROWS_PT
    nblk = (NBLK - s + (NS - 1)) // NS
    npair = nblk // 2

    sidx = [sidx0, sidx1]
    didx = [didx0, didx1]
    rows = [rows0, rows1]
    sg = [sg0, sg1]
    ss = [ss0, ss1]
    si = [si0, si1]

    def g0_of(k):
        return (s + k * NS) * GPB

    def load_idx(sl, k):
        g0 = g0_of(k)
        pltpu.async_copy(src_hbm.at[pl.ds(g0, GPB)], sidx[sl], si[sl])
        pltpu.async_copy(dst_hbm.at[pl.ds(g0, GPB)], didx[sl], si[sl])

    def wait_idx(sl, k):
        g0 = g0_of(k)
        pltpu.make_async_copy(
            src_hbm.at[pl.ds(g0, GPB)], sidx[sl], si[sl]).wait()
        pltpu.make_async_copy(
            dst_hbm.at[pl.ds(g0, GPB)], didx[sl], si[sl]).wait()

    def adjust(sl, slab):
        for j in range(GPB):
            def adj(q, _):
                v = sidx[sl][j, pl.ds(q * LANES, LANES)]
                sidx[sl][j, pl.ds(q * LANES, LANES)] = v * 8 + slab
                return 0
            lax.fori_loop(0, GP // LANES, adj, 0)

    def gathers(sl):
        for j in range(GPB):
            pltpu.async_copy(p_hbm.at[sidx[sl].at[j]],
                             rows[sl].at[pl.ds(j * GP, GP)], sg[sl])

    def wait_gathers(sl):
        for j in range(GPB):
            pltpu.make_async_copy(p_hbm.at[sidx[sl].at[j]],
                                  rows[sl].at[pl.ds(j * GP, GP)],
                                  sg[sl]).wait()

    def scatters(sl):
        for j in range(GPB):
            pltpu.async_copy(rows[sl].at[pl.ds(j * GP, GP)],
                             accum.at[didx[sl].at[j]], ss[sl], add=True)

    def wait_scatters(sl):
        for j in range(GPB):
            pltpu.make_async_copy(rows[sl].at[pl.ds(j * GP, GP)],
                                  accum.at[didx[sl].at[j]], ss[sl]).wait()

    for t in range(SPC):
        slab = c * SPC + t

        pltpu.sync_copy(zeros_hbm, zbuf)
        for r in range(ROWS_PT // ZR):
            pltpu.sync_copy(zbuf, accum.at[pl.ds(base + r * ZR, ZR)])
        plsc.subcore_barrier()

        load_idx(0, 0)
        wait_idx(0, 0)
        adjust(0, slab)
        gathers(0)
        load_idx(1, 1)

        def pair(i, _):
            b1 = 2 * i + 1
            b2 = 2 * i + 2

            @pl.when(i > 0)
            def _():
                wait_scatters(1)
                load_idx(1, b1)

            wait_idx(1, b1)
            adjust(1, slab)
            wait_gathers(0)
            scatters(0)
            gathers(1)
            wait_scatters(0)

            @pl.when(b2 < nblk)
            def _():
                load_idx(0, b2)
                wait_idx(0, b2)
                adjust(0, slab)
                gathers(0)

            wait_gathers(1)
            scatters(1)
            return 0

        lax.fori_loop(0, npair, pair, 0)
        wait_scatters(1)

        @pl.when(nblk % 2 == 1)
        def _():
            wait_gathers(0)
            scatters(0)
            wait_scatters(0)

        plsc.subcore_barrier()
        for r in range(ROWS_PT // ZR):
            pltpu.sync_copy(accum.at[pl.ds(base + r * ZR, ZR)], zbuf)
            pltpu.sync_copy(zbuf, out_hbm.at[pl.ds(base + r * ZR, ZR),
                                             pl.ds(slab * SD, SD)])
        plsc.subcore_barrier()


def _sc_edge_agg(p_flat, src, dst, zeros_v, nslab):
    mesh = plsc.VectorSubcoreMesh(core_axis_name="c", subcore_axis_name="s",
                                  num_cores=NC, num_subcores=NS)
    k = pl.kernel(
        functools.partial(_sc_agg_body, nslab // NC),
        out_type=jax.ShapeDtypeStruct((NPAD, 128), jnp.float32),
        mesh=mesh,
        compiler_params=pltpu.CompilerParams(use_tc_tiling_on_sc=False),
        scratch_types=[
            pltpu.VMEM((GPB, GP), jnp.int32),
            pltpu.VMEM((GPB, GP), jnp.int32),
            pltpu.VMEM((GPB, GP), jnp.int32),
            pltpu.VMEM((GPB, GP), jnp.int32),
            pltpu.VMEM((BLK, SD), jnp.float32),
            pltpu.VMEM((BLK, SD), jnp.float32),
            pltpu.VMEM((ZR, SD), jnp.float32),
            pltpu.VMEM_SHARED((NPAD, SD), jnp.float32),
            pltpu.SemaphoreType.DMA,
            pltpu.SemaphoreType.DMA,
            pltpu.SemaphoreType.DMA,
            pltpu.SemaphoreType.DMA,
            pltpu.SemaphoreType.DMA,
            pltpu.SemaphoreType.DMA,
        ],
    )
    return k(p_flat, src, dst, zeros_v)


def _dcol_from_degp(dp):
    deg = dp[0] + dp[1] + 1.0
    dinv = lax.rsqrt(deg)
    ident = (lax.broadcasted_iota(jnp.int32, (128, 128), 0) ==
             lax.broadcasted_iota(jnp.int32, (128, 128), 1)
             ).astype(jnp.float32)
    cols = [lax.dot_general(ident, dinv[r:r + 1, :],
                            (((1,), (1,)), ((), ())),
                            preferred_element_type=jnp.float32)
            for r in range(RB // 128)]
    return jnp.concatenate(cols, axis=0)


def _tc_k2_body(x_ref, w_ref, dp_ref, p_ref):
    dcol = _dcol_from_degp(dp_ref[...])
    h = jnp.dot(x_ref[...], w_ref[...], preferred_element_type=jnp.float32)
    p = h * dcol
    p_ref[...] = jnp.concatenate(
        [p, jnp.zeros((RB, 128 - HID_CH), jnp.float32)], axis=1)


def _tc_k2(xp, W1, degp):
    return pl.pallas_call(
        _tc_k2_body,
        grid=(NRB,),
        in_specs=[
            pl.BlockSpec((RB, IN_CH), lambda i: (i, 0)),
            pl.BlockSpec((IN_CH, HID_CH), lambda i: (0, 0)),
            pl.BlockSpec((NC, RB // 128, 128), lambda i: (0, i, 0)),
        ],
        out_specs=pl.BlockSpec((RB, 128), lambda i: (i, 0)),
        out_shape=jax.ShapeDtypeStruct((NPAD, 128), jnp.float32),
    )(xp, W1, degp)


def _tc_k4_body(a_ref, p_ref, d_ref, b1_ref, w2_ref, o_ref):
    dcol = _dcol_from_degp(d_ref[...])
    acc = a_ref[...][:, :HID_CH]
    p1 = p_ref[...][:, :HID_CH]
    h1 = jnp.maximum((acc + p1) * dcol + b1_ref[0:1, :], 0.0)
    m = jnp.dot(h1, w2_ref[...], preferred_element_type=jnp.float32)
    p2 = m * dcol
    o_ref[...] = jnp.concatenate(
        [p2, jnp.zeros((RB, 128 - OUT_CH), jnp.float32)], axis=1)


def _tc_k4(accum1, P1, dinv, b1r, W2):
    return pl.pallas_call(
        _tc_k4_body,
        grid=(NRB,),
        in_specs=[
            pl.BlockSpec((RB, 128), lambda i: (i, 0)),
            pl.BlockSpec((RB, 128), lambda i: (i, 0)),
            pl.BlockSpec((NC, RB // 128, 128), lambda i: (0, i, 0)),
            pl.BlockSpec((8, HID_CH), lambda i: (0, 0)),
            pl.BlockSpec((HID_CH, 32), lambda i: (0, 0)),
        ],
        out_specs=pl.BlockSpec((RB, 128), lambda i: (i, 0)),
        out_shape=jax.ShapeDtypeStruct((NPAD, 128), jnp.float32),
    )(accum1, P1, dinv, b1r, W2)


def _tc_k6_body(a_ref, p_ref, d_ref, b2_ref, batch_ref, sum_ref, cnt_ref):
    i = pl.program_id(0)
    dcol = _dcol_from_degp(d_ref[...])
    acc = a_ref[...][:, :OUT_CH]
    p2 = p_ref[...][:, :OUT_CH]
    h2 = (acc + p2) * dcol + b2_ref[0:1, :]
    bv = batch_ref[0, 0, :]
    seg = lax.broadcasted_iota(jnp.int32, (NUM_GRAPHS, RB), 0)
    oh = (seg == bv[None, :]).astype(jnp.float32)
    part = jnp.dot(oh, h2, preferred_element_type=jnp.float32)
    pcnt = jnp.sum(oh, axis=1, keepdims=True)

    @pl.when(i == 0)
    def _():
        sum_ref[...] = jnp.zeros_like(sum_ref)
        cnt_ref[...] = jnp.zeros_like(cnt_ref)

    sum_ref[...] += part
    cnt_ref[...] += pcnt


def _tc_k6(accum2, P2, dinv, b2r, batch3):
    return pl.pallas_call(
        _tc_k6_body,
        grid=(NRB,),
        in_specs=[
            pl.BlockSpec((RB, 128), lambda i: (i, 0)),
            pl.BlockSpec((RB, 128), lambda i: (i, 0)),
            pl.BlockSpec((NC, RB // 128, 128), lambda i: (0, i, 0)),
            pl.BlockSpec((8, 32), lambda i: (0, 0)),
            pl.BlockSpec((1, 1, RB), lambda i: (i, 0, 0)),
        ],
        out_specs=[
            pl.BlockSpec((NUM_GRAPHS, 32), lambda i: (0, 0)),
            pl.BlockSpec((NUM_GRAPHS, 1), lambda i: (0, 0)),
        ],
        out_shape=[
            jax.ShapeDtypeStruct((NUM_GRAPHS, 32), jnp.float32),
            jax.ShapeDtypeStruct((NUM_GRAPHS, 1), jnp.float32),
        ],
    )(accum2, P2, dinv, b2r, batch3)


def _tc_k7_body(s_ref, c_ref, w1_ref, b1_ref, w2_ref, b2_ref, o_ref):
    pooled = s_ref[...] / jnp.maximum(c_ref[...], 1.0)
    t = jnp.maximum(
        jnp.dot(pooled, w1_ref[...], preferred_element_type=jnp.float32)
        + b1_ref[0:1, :], 0.0)
    o_ref[...] = (jnp.dot(t, w2_ref[...], preferred_element_type=jnp.float32)
                  + b2_ref[0:1, :])


def _tc_k7(sums, cnt, fc1_W, fc1_br, fc2_W, fc2_br):
    return pl.pallas_call(
        _tc_k7_body,
        out_shape=jax.ShapeDtypeStruct((NUM_GRAPHS, OUT_CH), jnp.float32),
    )(sums, cnt, fc1_W, fc1_br, fc2_W, fc2_br)


def kernel(x, edge_index, batch, W1, b1, W2, b2, fc1_W, fc1_b, fc2_W, fc2_b):
    src = jnp.pad(edge_index[0], (0, EPAD - E)).reshape(NBLK * GPB, GP)
    dst = jnp.pad(edge_index[1], (0, EPAD - E),
                  constant_values=N).reshape(NBLK * GPB, GP)

    xp = jnp.pad(x, ((0, NPAD - N), (0, 0)))
    batchp = jnp.pad(batch, (0, NPAD - N),
                     constant_values=NUM_GRAPHS).reshape(NRB, 1, RB)
    ones_v = jnp.ones((GP,), jnp.float32)
    zeros1 = jnp.zeros((ZR,), jnp.float32)
    zeros16 = jnp.zeros((ZR, SD), jnp.float32)
    b1r = jnp.broadcast_to(b1[None, :], (8, HID_CH))
    b2r = jnp.broadcast_to(b2[None, :], (8, OUT_CH))
    fc1_br = jnp.broadcast_to(fc1_b[None, :], (8, OUT_CH))
    fc2_br = jnp.broadcast_to(fc2_b[None, :], (8, OUT_CH))

    degp = _sc_degree(dst, ones_v, zeros1).reshape(NC, NPAD // 128, 128)
    P1 = _tc_k2(xp, W1, degp)
    accum1 = _sc_edge_agg(P1.reshape(8 * NPAD, SD), src, dst, zeros16, 4)
    P2 = _tc_k4(accum1, P1, degp, b1r, W2)
    accum2 = _sc_edge_agg(P2.reshape(8 * NPAD, SD), src, dst, zeros16, 2)
    sums, cnt = _tc_k6(accum2, P2, degp, b2r, batchp)
    return _tc_k7(sums, cnt, fc1_W, fc1_br, fc2_W, fc2_br)


# speedup vs baseline: 51.5130x; 1.0685x over previous
import functools

import jax
import jax.numpy as jnp
from jax import lax
from jax.experimental import pallas as pl
from jax.experimental.pallas import tpu as pltpu
from jax.experimental.pallas import tpu_sc as plsc

N = 50000
E = 800000
IN_CH = 128
HID_CH = 64
OUT_CH = 32
NUM_GRAPHS = 256

NC, NS, LANES = 2, 16, 16
NPAD = 50176
ROWS_PT = NPAD // NS
GP = 128
GPB = 16
BLK = GP * GPB
NGRP = E // GP
NBLK = 390
ZR = 448
SD = 16
RB = 1024
NRB = NPAD // RB


def _sc_degree_body(eidx_hbm, tail_hbm, ones_hbm, zeros_hbm, out_hbm,
                    didx, ones_v, zbuf, hist, sem):
    c = lax.axis_index("c")
    s = lax.axis_index("s")
    wid = c * NS + s

    pltpu.sync_copy(zeros_hbm, zbuf)
    base = s * ROWS_PT
    for r in range(ROWS_PT // ZR):
        pltpu.sync_copy(zbuf, hist.at[pl.ds(base + r * ZR, ZR)])
    pltpu.sync_copy(ones_hbm, ones_v)
    plsc.subcore_barrier()

    nblk = (NBLK - wid + (NC * NS - 1)) // (NC * NS)

    def blk(k, _):
        g0 = (wid + k * NC * NS) * GPB
        pltpu.sync_copy(eidx_hbm.at[1].at[pl.ds(g0, GPB)], didx)
        descs = [
            pltpu.async_copy(ones_v, hist.at[didx.at[j]], sem, add=True)
            for j in range(GPB)
        ]
        for d in descs:
            d.wait()
        return 0

    lax.fori_loop(0, nblk, blk, 0)

    @pl.when(c == 0)
    def _():
        pltpu.sync_copy(tail_hbm.at[1], didx)
        d = pltpu.async_copy(ones_v, hist.at[didx.at[s]], sem, add=True)
        d.wait()

    plsc.subcore_barrier()
    for r in range(ROWS_PT // ZR):
        pltpu.sync_copy(hist.at[pl.ds(base + r * ZR, ZR)], zbuf)
        pltpu.sync_copy(zbuf, out_hbm.at[pl.ds(c * NPAD + base + r * ZR, ZR)])


def _sc_degree(eidx, tail, ones_v, zeros_v):
    mesh = plsc.VectorSubcoreMesh(core_axis_name="c", subcore_axis_name="s",
                                  num_cores=NC, num_subcores=NS)
    k = pl.kernel(
        _sc_degree_body,
        out_type=jax.ShapeDtypeStruct((NC * NPAD,), jnp.float32),
        mesh=mesh,
        scratch_types=[
            pltpu.VMEM((GPB, GP), jnp.int32),
            pltpu.VMEM((GP,), jnp.float32),
            pltpu.VMEM((ZR,), jnp.float32),
            pltpu.VMEM_SHARED((NPAD,), jnp.float32),
            pltpu.SemaphoreType.DMA,
        ],
    )
    return k(eidx, tail, ones_v, zeros_v)


def _sc_agg_body(SPC, p_hbm, eidx_hbm, tail_hbm, zeros_hbm, out_hbm,
                 sidx0, didx0, sidx1, didx1, rows0, rows1, zbuf, accum,
                 sg0, sg1, ss0, ss1, si0, si1):
    c = lax.axis_index("c")
    s = lax.axis_index("s")
    base = s * ROWS_PT
    nblk = (NBLK - s + (NS - 1)) // NS
    npair = nblk // 2

    sidx = [sidx0, sidx1]
    didx = [didx0, didx1]
    rows = [rows0, rows1]
    sg = [sg0, sg1]
    ss = [ss0, ss1]
    si = [si0, si1]

    def g0_of(k):
        return (s + k * NS) * GPB

    def load_idx(sl, k):
        g0 = g0_of(k)
        pltpu.async_copy(eidx_hbm.at[0].at[pl.ds(g0, GPB)], sidx[sl], si[sl])
        pltpu.async_copy(eidx_hbm.at[1].at[pl.ds(g0, GPB)], didx[sl], si[sl])

    def wait_idx(sl, k):
        g0 = g0_of(k)
        pltpu.make_async_copy(
            eidx_hbm.at[0].at[pl.ds(g0, GPB)], sidx[sl], si[sl]).wait()
        pltpu.make_async_copy(
            eidx_hbm.at[1].at[pl.ds(g0, GPB)], didx[sl], si[sl]).wait()

    def adjust(sl, slab):
        for j in range(GPB):
            def adj(q, _):
                v = sidx[sl][j, pl.ds(q * LANES, LANES)]
                sidx[sl][j, pl.ds(q * LANES, LANES)] = v * 8 + slab
                return 0
            lax.fori_loop(0, GP // LANES, adj, 0)

    def gathers(sl):
        for j in range(GPB):
            pltpu.async_copy(p_hbm.at[sidx[sl].at[j]],
                             rows[sl].at[pl.ds(j * GP, GP)], sg[sl])

    def wait_gathers(sl):
        for j in range(GPB):
            pltpu.make_async_copy(p_hbm.at[sidx[sl].at[j]],
                                  rows[sl].at[pl.ds(j * GP, GP)],
                                  sg[sl]).wait()

    def scatters(sl):
        for j in range(GPB):
            pltpu.async_copy(rows[sl].at[pl.ds(j * GP, GP)],
                             accum.at[didx[sl].at[j]], ss[sl], add=True)

    def wait_scatters(sl):
        for j in range(GPB):
            pltpu.make_async_copy(rows[sl].at[pl.ds(j * GP, GP)],
                                  accum.at[didx[sl].at[j]], ss[sl]).wait()

    for t in range(SPC):
        slab = c * SPC + t

        pltpu.sync_copy(zeros_hbm, zbuf)
        for r in range(ROWS_PT // ZR):
            pltpu.sync_copy(zbuf, accum.at[pl.ds(base + r * ZR, ZR)])
        plsc.subcore_barrier()

        load_idx(0, 0)
        wait_idx(0, 0)
        adjust(0, slab)
        gathers(0)
        load_idx(1, 1)

        def pair(i, _):
            b1 = 2 * i + 1
            b2 = 2 * i + 2

            @pl.when(i > 0)
            def _():
                wait_scatters(1)
                load_idx(1, b1)

            wait_idx(1, b1)
            adjust(1, slab)
            wait_gathers(0)
            scatters(0)
            gathers(1)
            wait_scatters(0)

            @pl.when(b2 < nblk)
            def _():
                load_idx(0, b2)
                wait_idx(0, b2)
                adjust(0, slab)
                gathers(0)

            wait_gathers(1)
            scatters(1)
            return 0

        lax.fori_loop(0, npair, pair, 0)
        wait_scatters(1)

        @pl.when(nblk % 2 == 1)
        def _():
            wait_gathers(0)
            scatters(0)
            wait_scatters(0)

        pltpu.sync_copy(tail_hbm.at[0], sidx0)
        pltpu.sync_copy(tail_hbm.at[1], didx0)
        adjust(0, slab)
        pltpu.async_copy(p_hbm.at[sidx0.at[s]],
                         rows0.at[pl.ds(0, GP)], sg0)
        pltpu.make_async_copy(p_hbm.at[sidx0.at[s]],
                              rows0.at[pl.ds(0, GP)], sg0).wait()
        pltpu.async_copy(rows0.at[pl.ds(0, GP)],
                         accum.at[didx0.at[s]], ss0, add=True)
        pltpu.make_async_copy(rows0.at[pl.ds(0, GP)],
                              accum.at[didx0.at[s]], ss0).wait()

        plsc.subcore_barrier()
        for r in range(ROWS_PT // ZR):
            pltpu.sync_copy(accum.at[pl.ds(base + r * ZR, ZR)], zbuf)
            pltpu.sync_copy(zbuf, out_hbm.at[pl.ds(base + r * ZR, ZR),
                                             pl.ds(slab * SD, SD)])
        plsc.subcore_barrier()


def _sc_edge_agg(p_flat, eidx, tail, zeros_v, nslab):
    mesh = plsc.VectorSubcoreMesh(core_axis_name="c", subcore_axis_name="s",
                                  num_cores=NC, num_subcores=NS)
    k = pl.kernel(
        functools.partial(_sc_agg_body, nslab // NC),
        out_type=jax.ShapeDtypeStruct((NPAD, 128), jnp.float32),
        mesh=mesh,
        compiler_params=pltpu.CompilerParams(use_tc_tiling_on_sc=False),
        scratch_types=[
            pltpu.VMEM((GPB, GP), jnp.int32),
            pltpu.VMEM((GPB, GP), jnp.int32),
            pltpu.VMEM((GPB, GP), jnp.int32),
            pltpu.VMEM((GPB, GP), jnp.int32),
            pltpu.VMEM((BLK, SD), jnp.float32),
            pltpu.VMEM((BLK, SD), jnp.float32),
            pltpu.VMEM((ZR, SD), jnp.float32),
            pltpu.VMEM_SHARED((NPAD, SD), jnp.float32),
            pltpu.SemaphoreType.DMA,
            pltpu.SemaphoreType.DMA,
            pltpu.SemaphoreType.DMA,
            pltpu.SemaphoreType.DMA,
            pltpu.SemaphoreType.DMA,
            pltpu.SemaphoreType.DMA,
        ],
    )
    return k(p_flat, eidx, tail, zeros_v)


def _dcol_from_degp(dp):
    deg = dp[0] + dp[1] + 1.0
    dinv = lax.rsqrt(deg)
    ident = (lax.broadcasted_iota(jnp.int32, (128, 128), 0) ==
             lax.broadcasted_iota(jnp.int32, (128, 128), 1)
             ).astype(jnp.float32)
    cols = [lax.dot_general(ident, dinv[r:r + 1, :],
                            (((1,), (1,)), ((), ())),
                            preferred_element_type=jnp.float32)
            for r in range(RB // 128)]
    return jnp.concatenate(cols, axis=0)


def _tc_k2_body(x_ref, w_ref, dp_ref, p_ref):
    i = pl.program_id(0)
    dcol = _dcol_from_degp(dp_ref[...])
    h = jnp.dot(x_ref[...], w_ref[...], preferred_element_type=jnp.float32)
    p = h * dcol
    rowid = i * RB + lax.broadcasted_iota(jnp.int32, (RB, 1), 0)
    p = jnp.where(rowid < N, p, 0.0)
    p_ref[...] = jnp.concatenate(
        [p, jnp.zeros((RB, 128 - HID_CH), jnp.float32)], axis=1)


def _tc_k2(xp, W1, degp):
    return pl.pallas_call(
        _tc_k2_body,
        grid=(NRB,),
        in_specs=[
            pl.BlockSpec((RB, IN_CH), lambda i: (i, 0)),
            pl.BlockSpec((IN_CH, HID_CH), lambda i: (0, 0)),
            pl.BlockSpec((NC, RB // 128, 128), lambda i: (0, i, 0)),
        ],
        out_specs=pl.BlockSpec((RB, 128), lambda i: (i, 0)),
        out_shape=jax.ShapeDtypeStruct((NPAD, 128), jnp.float32),
    )(xp, W1, degp)


def _tc_k4_body(a_ref, p_ref, d_ref, b1_ref, w2_ref, o_ref):
    dcol = _dcol_from_degp(d_ref[...])
    acc = a_ref[...][:, :HID_CH]
    p1 = p_ref[...][:, :HID_CH]
    h1 = jnp.maximum((acc + p1) * dcol + b1_ref[0:1, :], 0.0)
    m = jnp.dot(h1, w2_ref[...], preferred_element_type=jnp.float32)
    p2 = m * dcol
    o_ref[...] = jnp.concatenate(
        [p2, jnp.zeros((RB, 128 - OUT_CH), jnp.float32)], axis=1)


def _tc_k4(accum1, P1, dinv, b1r, W2):
    return pl.pallas_call(
        _tc_k4_body,
        grid=(NRB,),
        in_specs=[
            pl.BlockSpec((RB, 128), lambda i: (i, 0)),
            pl.BlockSpec((RB, 128), lambda i: (i, 0)),
            pl.BlockSpec((NC, RB // 128, 128), lambda i: (0, i, 0)),
            pl.BlockSpec((8, HID_CH), lambda i: (0, 0)),
            pl.BlockSpec((HID_CH, 32), lambda i: (0, 0)),
        ],
        out_specs=pl.BlockSpec((RB, 128), lambda i: (i, 0)),
        out_shape=jax.ShapeDtypeStruct((NPAD, 128), jnp.float32),
    )(accum1, P1, dinv, b1r, W2)


def _tc_k6_body(a_ref, p_ref, d_ref, b2_ref, batch_ref, sum_ref, cnt_ref):
    i = pl.program_id(0)
    dcol = _dcol_from_degp(d_ref[...])
    acc = a_ref[...][:, :OUT_CH]
    p2 = p_ref[...][:, :OUT_CH]
    h2 = (acc + p2) * dcol + b2_ref[0:1, :]
    bv = batch_ref[0, 0, :]
    seg = lax.broadcasted_iota(jnp.int32, (NUM_GRAPHS, RB), 0)
    oh = (seg == bv[None, :]).astype(jnp.float32)
    part = jnp.dot(oh, h2, preferred_element_type=jnp.float32)
    pcnt = jnp.sum(oh, axis=1, keepdims=True)

    @pl.when(i == 0)
    def _():
        sum_ref[...] = jnp.zeros_like(sum_ref)
        cnt_ref[...] = jnp.zeros_like(cnt_ref)

    sum_ref[...] += part
    cnt_ref[...] += pcnt


def _tc_k6(accum2, P2, dinv, b2r, batch3):
    return pl.pallas_call(
        _tc_k6_body,
        grid=(NRB,),
        in_specs=[
            pl.BlockSpec((RB, 128), lambda i: (i, 0)),
            pl.BlockSpec((RB, 128), lambda i: (i, 0)),
            pl.BlockSpec((NC, RB // 128, 128), lambda i: (0, i, 0)),
            pl.BlockSpec((8, 32), lambda i: (0, 0)),
            pl.BlockSpec((1, 1, RB), lambda i: (i, 0, 0)),
        ],
        out_specs=[
            pl.BlockSpec((NUM_GRAPHS, 32), lambda i: (0, 0)),
            pl.BlockSpec((NUM_GRAPHS, 1), lambda i: (0, 0)),
        ],
        out_shape=[
            jax.ShapeDtypeStruct((NUM_GRAPHS, 32), jnp.float32),
            jax.ShapeDtypeStruct((NUM_GRAPHS, 1), jnp.float32),
        ],
    )(accum2, P2, dinv, b2r, batch3)


def _tc_k7_body(s_ref, c_ref, w1_ref, b1_ref, w2_ref, b2_ref, o_ref):
    pooled = s_ref[...] / jnp.maximum(c_ref[...], 1.0)
    t = jnp.maximum(
        jnp.dot(pooled, w1_ref[...], preferred_element_type=jnp.float32)
        + b1_ref[0:1, :], 0.0)
    o_ref[...] = (jnp.dot(t, w2_ref[...], preferred_element_type=jnp.float32)
                  + b2_ref[0:1, :])


def _tc_k7(sums, cnt, fc1_W, fc1_br, fc2_W, fc2_br):
    return pl.pallas_call(
        _tc_k7_body,
        out_shape=jax.ShapeDtypeStruct((NUM_GRAPHS, OUT_CH), jnp.float32),
    )(sums, cnt, fc1_W, fc1_br, fc2_W, fc2_br)


def kernel(x, edge_index, batch, W1, b1, W2, b2, fc1_W, fc1_b, fc2_W, fc2_b):
    eidx = edge_index.reshape(2, NGRP, GP)
    t0 = NBLK * GPB * GP
    ts = jnp.pad(edge_index[0, t0:].reshape(NGRP - NBLK * GPB, GP),
                 ((0, 6), (0, 0)))
    td = jnp.pad(edge_index[1, t0:].reshape(NGRP - NBLK * GPB, GP),
                 ((0, 6), (0, 0)), constant_values=N)
    tail = jnp.stack([ts, td])

    batchp = jnp.pad(batch, (0, NPAD - N),
                     constant_values=NUM_GRAPHS).reshape(NRB, 1, RB)
    ones_v = jnp.ones((GP,), jnp.float32)
    zeros1 = jnp.zeros((ZR,), jnp.float32)
    zeros16 = jnp.zeros((ZR, SD), jnp.float32)
    b1r = jnp.broadcast_to(b1[None, :], (8, HID_CH))
    b2r = jnp.broadcast_to(b2[None, :], (8, OUT_CH))
    fc1_br = jnp.broadcast_to(fc1_b[None, :], (8, OUT_CH))
    fc2_br = jnp.broadcast_to(fc2_b[None, :], (8, OUT_CH))

    degp = _sc_degree(eidx, tail, ones_v, zeros1).reshape(
        NC, NPAD // 128, 128)
    P1 = _tc_k2(x, W1, degp)
    accum1 = _sc_edge_agg(P1.reshape(8 * NPAD, SD), eidx, tail, zeros16, 4)
    P2 = _tc_k4(accum1, P1, degp, b1r, W2)
    accum2 = _sc_edge_agg(P2.reshape(8 * NPAD, SD), eidx, tail, zeros16, 2)
    sums, cnt = _tc_k6(accum2, P2, degp, b2r, batchp)
    return _tc_k7(sums, cnt, fc1_W, fc1_br, fc2_W, fc2_br)
